# baseline (speedup 1.0000x reference)
import jax
import jax.numpy as jnp
from jax import lax
from jax.experimental import pallas as pl
from jax.experimental.pallas import tpu as pltpu
from jax.experimental.pallas import tpu_sc as plsc

VOCAB = 1000000
EMBED_DIM = 64
BATCH = 16
SEQ_LEN = 2048

NC = 2
NS = 16
NW = NC * NS
S_PER_W = SEQ_LEN // NW
HALF_B = BATCH // 2
HALF_ROWS = HALF_B * S_PER_W
HALF_PAIRS = HALF_ROWS // 2
LANES = 16
PAIR = 2 * EMBED_DIM
NSTREAM = HALF_ROWS // PAIR


def _body(idx_hbm, table_hbm, pos_hbm, out_hbm,
          idx_v, rows_v, outc_v, pos_v, sem):
    wid = lax.axis_index("s") * NC + lax.axis_index("c")
    s0 = pl.multiple_of(wid * S_PER_W, S_PER_W)
    p0 = pl.multiple_of(wid * (S_PER_W // 2), S_PER_W // 2)

    pltpu.sync_copy(pos_hbm.at[pl.ds(p0, S_PER_W // 2)], pos_v)

    for h in range(2):
        for b in range(HALF_B):
            pltpu.sync_copy(
                idx_hbm.at[h * HALF_B + b, pl.ds(s0, S_PER_W)],
                idx_v.at[pl.ds(b * S_PER_W, S_PER_W)])

        NGROUP = HALF_ROWS // LANES
        WINDOW = 8

        def fire_group(g):
            vec = idx_v[pl.ds(g * LANES, LANES)]
            for j in range(LANES):
                v = vec[j]
                pltpu.async_copy(
                    table_hbm.at[pl.ds(v >> 1, 1)],
                    rows_v.at[pl.ds(g * LANES + j, 1)],
                    sem)

        def drain_rows(n):
            pltpu.make_async_copy(
                table_hbm.at[pl.ds(0, n)], rows_v.at[pl.ds(0, n)], sem
            ).wait()

        for g in range(WINDOW):
            fire_group(g)

        def gather_step(g, _):
            fire_group(g)
            drain_rows(LANES)
            return 0

        lax.fori_loop(WINDOW, NGROUP, gather_step, 0)
        drain_rows(WINDOW * LANES)

        def add_pair(p, _):
            iv = idx_v[pl.ds(2 * p, LANES)]
            i_half = p % (S_PER_W // 2)
            for rr in range(2):
                r2 = 2 * p + rr
                hr = iv[rr] & 1
                for j in range(EMBED_DIM // LANES):
                    v0 = rows_v[r2, pl.ds(j * LANES, LANES)]
                    v1 = rows_v[r2, pl.ds(EMBED_DIM + j * LANES, LANES)]
                    pv = pos_v[i_half, pl.ds(rr * EMBED_DIM + j * LANES,
                                             LANES)]
                    outc_v[p, pl.ds(rr * EMBED_DIM + j * LANES, LANES)] = (
                        jnp.where(hr == 1, v1, v0) + pv)
            return 0

        lax.fori_loop(0, HALF_PAIRS, add_pair, 0)

        for b in range(HALF_B):
            pltpu.sync_copy(
                outc_v.at[pl.ds(b * (S_PER_W // 2), S_PER_W // 2)],
                out_hbm.at[h * HALF_B + b, pl.ds(p0, S_PER_W // 2)])


@jax.jit
def _embed(token_indices, table2, pos2):
    mesh = plsc.VectorSubcoreMesh(core_axis_name="c", subcore_axis_name="s")
    run = pl.kernel(
        _body,
        out_type=jax.ShapeDtypeStruct((BATCH, SEQ_LEN // 2, PAIR),
                                      jnp.float32),
        mesh=mesh,
        scratch_types=[
            pltpu.VMEM((HALF_ROWS + LANES,), jnp.int32),
            pltpu.VMEM((HALF_ROWS, PAIR), jnp.float32),
            pltpu.VMEM((HALF_PAIRS, PAIR), jnp.float32),
            pltpu.VMEM((S_PER_W // 2, PAIR), jnp.float32),
            pltpu.SemaphoreType.DMA,
        ],
        compiler_params=pltpu.CompilerParams(use_tc_tiling_on_sc=True),
    )
    return run(token_indices, table2, pos2)


def kernel(token_indices, token_table, pos_table):
    table2 = token_table.reshape(VOCAB // 2, PAIR)
    pos2 = pos_table.reshape(SEQ_LEN // 2, PAIR)
    out = _embed(token_indices.astype(jnp.int32), table2, pos2)
    return out.reshape(BATCH, SEQ_LEN, EMBED_DIM)


# speedup vs baseline: 1.7342x; 1.7342x over previous
import jax
import jax.numpy as jnp
from jax import lax
from jax.experimental import pallas as pl
from jax.experimental.pallas import tpu as pltpu
from jax.experimental.pallas import tpu_sc as plsc

VOCAB = 1000000
EMBED_DIM = 64
BATCH = 16
SEQ_LEN = 2048

NC = 2
NS = 16
NW = NC * NS
S_PER_W = SEQ_LEN // NW
HALF_B = BATCH // 2
HALF_ROWS = HALF_B * S_PER_W
LANES = 16


def _body(idx_hbm, table_hbm, pos_hbm, out_hbm, idx_v, rows_v, pos_v, sem):
    wid = lax.axis_index("s") * NC + lax.axis_index("c")
    s0 = wid * S_PER_W

    pltpu.sync_copy(pos_hbm.at[pl.ds(s0, S_PER_W)], pos_v)

    for h in range(2):
        for b in range(HALF_B):
            pltpu.sync_copy(
                idx_hbm.at[h * HALF_B + b, pl.ds(s0, S_PER_W)],
                idx_v.at[pl.ds(b * S_PER_W, S_PER_W)])

        NGROUP = HALF_ROWS // LANES
        WINDOW = 8

        def fire_group(g):
            vec = idx_v[pl.ds(g * LANES, LANES)]
            for j in range(LANES):
                v = vec[j]
                pltpu.async_copy(
                    table_hbm.at[pl.ds(v, 1)],
                    rows_v.at[pl.ds(g * LANES + j, 1)],
                    sem)

        def drain_rows(n):
            pltpu.make_async_copy(
                table_hbm.at[pl.ds(0, n)], rows_v.at[pl.ds(0, n)], sem
            ).wait()

        for g in range(WINDOW):
            fire_group(g)

        def gather_step(g, _):
            fire_group(g)
            drain_rows(LANES)
            return 0

        lax.fori_loop(WINDOW, NGROUP, gather_step, 0)
        drain_rows(WINDOW * LANES)

        def add_row(i, _):
            for j in range(EMBED_DIM // LANES):
                p = pos_v[i, pl.ds(j * LANES, LANES)]
                for b in range(HALF_B):
                    r = b * S_PER_W + i
                    rows_v[r, pl.ds(j * LANES, LANES)] = (
                        rows_v[r, pl.ds(j * LANES, LANES)] + p)
            return 0

        lax.fori_loop(0, S_PER_W, add_row, 0)

        for b in range(HALF_B):
            pltpu.sync_copy(
                rows_v.at[pl.ds(b * S_PER_W, S_PER_W)],
                out_hbm.at[h * HALF_B + b, pl.ds(s0, S_PER_W)])


@jax.jit
def _embed(token_indices, token_table, pos_table):
    mesh = plsc.VectorSubcoreMesh(core_axis_name="c", subcore_axis_name="s")
    run = pl.kernel(
        _body,
        out_type=jax.ShapeDtypeStruct((BATCH, SEQ_LEN, EMBED_DIM),
                                      jnp.float32),
        mesh=mesh,
        scratch_types=[
            pltpu.VMEM((HALF_ROWS,), jnp.int32),
            pltpu.VMEM((HALF_ROWS, EMBED_DIM), jnp.float32),
            pltpu.VMEM((S_PER_W, EMBED_DIM), jnp.float32),
            pltpu.SemaphoreType.DMA,
        ],
        compiler_params=pltpu.CompilerParams(use_tc_tiling_on_sc=True),
    )
    return run(token_indices, token_table, pos_table)


def kernel(token_indices, token_table, pos_table):
    return _embed(token_indices.astype(jnp.int32), token_table, pos_table)
